# Initial kernel scaffold; baseline (speedup 1.0000x reference)
import jax
import jax.numpy as jnp
from jax.experimental import pallas as pl


def kernel(x, edge_index, edge_weight, Wq1, bq1, Wk1, bk1, Wv1, bv1, Ws1, bs1, Wq2, bq2, Wk2, bk2, Wv2, bv2, Ws2, bs2):
    raise NotImplementedError("write your pallas kernel here")


# speedup vs baseline: 5.4174x; 5.4174x over previous
import functools
import math

import jax
import jax.numpy as jnp
from jax import lax
from jax.experimental import pallas as pl
from jax.experimental.pallas import tpu as pltpu
from jax.experimental.pallas import tpu_sc as plsc

N = 10000
E = 320000
D_IN = 128
C1 = 256
C2 = 64

NC = 2
NS = 16
L = 16
NW = NC * NS
EPW = E // NW
CH = 80
NCH = EPW // CH

F32 = jnp.float32


def _mesh():
    return plsc.VectorSubcoreMesh(core_axis_name="c", subcore_axis_name="s")


def _make_alpha_kernel(C):
    NJ = C // L

    def body(dst_hbm, src_hbm, q_hbm, k_hbm, alpha_hbm, wmax_hbm,
             didx, sidx, qb, kb, pb, ab, mxb, sem_q, sem_k):
        cid = lax.axis_index("c")
        sid = lax.axis_index("s")
        wid = sid * NC + cid
        ebase = wid * EPW

        def chunk_body(ch, mx):
            off = pl.multiple_of(ebase + ch * CH, 8)
            pltpu.sync_copy(dst_hbm.at[pl.ds(off, CH)], didx)
            pltpu.sync_copy(src_hbm.at[pl.ds(off, CH)], sidx)
            cq = pltpu.async_copy(q_hbm.at[didx], qb, sem_q)
            ck = pltpu.async_copy(k_hbm.at[sidx], kb, sem_k)
            cq.wait()
            ck.wait()

            def edge_body(e, carry):
                acc = qb[e, pl.ds(0, L)] * kb[e, pl.ds(0, L)]
                for j in range(1, NJ):
                    acc = acc + qb[e, pl.ds(j * L, L)] * kb[e, pl.ds(j * L, L)]
                pb[e, pl.ds(0, L)] = acc
                return carry

            lax.fori_loop(0, CH, edge_body, 0)

            def group_body(g, mx_in):
                rows = g * L + lax.iota(jnp.int32, L)
                a16 = plsc.load_gather(pb, [rows, jnp.zeros((L,), jnp.int32)])
                for j in range(1, L):
                    a16 = a16 + plsc.load_gather(
                        pb, [rows, jnp.full((L,), j, jnp.int32)])
                ab[pl.ds(g * L, L)] = a16
                return jnp.maximum(mx_in, a16)

            mx = lax.fori_loop(0, CH // L, group_body, mx)
            pltpu.sync_copy(ab, alpha_hbm.at[pl.ds(off, CH)])
            return mx

        mx0 = jnp.full((L,), -jnp.inf, F32)
        mx = lax.fori_loop(0, NCH, chunk_body, mx0)
        mxb[pl.ds(0, L)] = mx
        pltpu.sync_copy(mxb, wmax_hbm.at[wid])

    kern = pl.kernel(
        body,
        out_type=(jax.ShapeDtypeStruct((E,), F32),
                  jax.ShapeDtypeStruct((NW, L), F32)),
        mesh=_mesh(),
        compiler_params=pltpu.CompilerParams(needs_layout_passes=False,
                                             use_tc_tiling_on_sc=False),
        scratch_types=[
            pltpu.VMEM((CH,), jnp.int32),
            pltpu.VMEM((CH,), jnp.int32),
            pltpu.VMEM((CH, C), F32),
            pltpu.VMEM((CH, C), F32),
            pltpu.VMEM((CH, L), F32),
            pltpu.VMEM((CH,), F32),
            pltpu.VMEM((L,), F32),
            pltpu.SemaphoreType.DMA,
            pltpu.SemaphoreType.DMA,
        ],
    )
    return kern


def _make_agg_kernel(D, with_denom):
    CW = D + L if with_denom else D
    ZR = 40
    NBLK = N // ZR

    def body(dst_hbm, src_hbm, alpha_hbm, wmax_hbm, v_hbm, out_hbm,
             didx, sidx, vb, sb, ab, eb, mb, zb, acc_sh, sem_v):
        cid = lax.axis_index("c")
        sid = lax.axis_index("s")
        wid = sid * NC + cid
        ebase = wid * EPW

        def zfill(r, c):
            for j in range(CW // L):
                zb[r, pl.ds(j * L, L)] = jnp.zeros((L,), F32)
            return c
        lax.fori_loop(0, ZR, zfill, 0)

        def zcopy(t, c):
            blk = sid + t * NS

            @pl.when(blk < NBLK)
            def _():
                off = pl.multiple_of(blk * ZR, 8)
                pltpu.sync_copy(zb, acc_sh.at[pl.ds(off, ZR)])
            return c
        lax.fori_loop(0, (NBLK + NS - 1) // NS, zcopy, 0)

        pltpu.sync_copy(wmax_hbm, mb)

        def mred(i, m):
            return jnp.maximum(m, mb[i, pl.ds(0, L)])
        m16 = lax.fori_loop(0, NW, mred, jnp.full((L,), -jnp.inf, F32))
        gmax = jnp.max(m16)

        plsc.subcore_barrier()

        def chunk_body(ch, carry):
            off = pl.multiple_of(ebase + ch * CH, 8)
            pltpu.sync_copy(dst_hbm.at[pl.ds(off, CH)], didx)
            pltpu.sync_copy(src_hbm.at[pl.ds(off, CH)], sidx)
            cv = pltpu.async_copy(v_hbm.at[sidx], vb, sem_v)
            pltpu.sync_copy(alpha_hbm.at[pl.ds(off, CH)], ab)

            cv.wait()

            def group_body(g, c):
                a16 = ab[pl.ds(g * L, L)]
                e16 = jnp.exp(a16 - gmax)
                for l in range(L):
                    s = e16[l]
                    e = g * L + l
                    for j in range(D // L):
                        sb[e, pl.ds(j * L, L)] = vb[e, pl.ds(j * L, L)] * s
                    if with_denom:
                        sb[e, pl.ds(D, L)] = jnp.broadcast_to(s, (L,))
                return c
            lax.fori_loop(0, CH // L, group_body, 0)

            pltpu.sync_copy(sb, acc_sh.at[didx], add=True)
            return carry

        lax.fori_loop(0, NCH, chunk_body, 0)

        plsc.subcore_barrier()

        @pl.when(sid == 0)
        def _():
            pltpu.sync_copy(acc_sh, out_hbm.at[cid])

    kern = pl.kernel(
        body,
        out_type=jax.ShapeDtypeStruct((NC, N, CW), F32),
        mesh=_mesh(),
        compiler_params=pltpu.CompilerParams(needs_layout_passes=False,
                                             use_tc_tiling_on_sc=False),
        scratch_types=[
            pltpu.VMEM((CH,), jnp.int32),
            pltpu.VMEM((CH,), jnp.int32),
            pltpu.VMEM((CH, D), F32),
            pltpu.VMEM((CH, CW), F32),
            pltpu.VMEM((CH,), F32),
            pltpu.VMEM((CH,), F32),
            pltpu.VMEM((NW, L), F32),
            pltpu.VMEM((ZR, CW), F32),
            pltpu.VMEM_SHARED((N, CW), F32),
            pltpu.SemaphoreType.DMA,
        ],
    )
    return kern


_RB = 1000


def _proj1_body(x_ref, wq, bq, wk, bk, wv, bv, ws, bs,
                q_ref, k_ref, va_ref, vb_ref, s_ref):
    xb = jnp.nan_to_num(x_ref[...], nan=0.0)
    scale = 1.0 / math.sqrt(C1)
    q = (jnp.dot(xb, wq[...], preferred_element_type=F32) + bq[...]) * scale
    k = jnp.dot(xb, wk[...], preferred_element_type=F32) + bk[...]
    v = jnp.dot(xb, wv[...], preferred_element_type=F32) + bv[...]
    s = jnp.dot(xb, ws[...], preferred_element_type=F32) + bs[...]
    q_ref[...] = q
    k_ref[...] = k
    va_ref[...] = v[:, :D_IN]
    vb_ref[...] = v[:, D_IN:]
    s_ref[...] = s


def _proj1(x, Wq, bq, Wk, bk, Wv, bv, Ws, bs):
    grid = N // _RB
    full = lambda r, c: pl.BlockSpec((r, c), lambda i: (0, 0))
    row = lambda c: pl.BlockSpec((_RB, c), lambda i: (i, 0))
    return pl.pallas_call(
        _proj1_body,
        grid=(grid,),
        in_specs=[row(D_IN),
                  full(D_IN, C1), full(1, C1), full(D_IN, C1), full(1, C1),
                  full(D_IN, C1), full(1, C1), full(D_IN, C1), full(1, C1)],
        out_specs=[row(C1), row(C1), row(D_IN), row(D_IN), row(C1)],
        out_shape=[jax.ShapeDtypeStruct((N, C1), F32),
                   jax.ShapeDtypeStruct((N, C1), F32),
                   jax.ShapeDtypeStruct((N, D_IN), F32),
                   jax.ShapeDtypeStruct((N, D_IN), F32),
                   jax.ShapeDtypeStruct((N, C1), F32)],
    )(x, Wq, bq, Wk, bk, Wv, bv, Ws, bs)


def _proj2_body(aa_ref, ab_ref, s1_ref, wq, bq, wk, bk, wv, bv, ws, bs,
                q_ref, k_ref, v_ref, s_ref):
    accA = aa_ref[0] + aa_ref[1]
    accB = ab_ref[0] + ab_ref[1]
    den = jnp.maximum(accA[:, D_IN:D_IN + 1], 1e-16)
    h = jnp.concatenate([accA[:, :D_IN], accB], axis=1) / den + s1_ref[...]
    scale = 1.0 / math.sqrt(C2)
    q = (jnp.dot(h, wq[...], preferred_element_type=F32) + bq[...]) * scale
    k = jnp.dot(h, wk[...], preferred_element_type=F32) + bk[...]
    v = jnp.dot(h, wv[...], preferred_element_type=F32) + bv[...]
    s = jnp.dot(h, ws[...], preferred_element_type=F32) + bs[...]
    q_ref[...] = q
    k_ref[...] = k
    v_ref[...] = v
    s_ref[...] = s


def _proj2(accA, accB, s1, Wq, bq, Wk, bk, Wv, bv, Ws, bs):
    grid = N // _RB
    full = lambda r, c: pl.BlockSpec((r, c), lambda i: (0, 0))
    row = lambda c: pl.BlockSpec((_RB, c), lambda i: (i, 0))
    acc = lambda c: pl.BlockSpec((NC, _RB, c), lambda i: (0, i, 0))
    return pl.pallas_call(
        _proj2_body,
        grid=(grid,),
        in_specs=[acc(D_IN + L), acc(D_IN), row(C1),
                  full(C1, C2), full(1, C2), full(C1, C2), full(1, C2),
                  full(C1, C2), full(1, C2), full(C1, C2), full(1, C2)],
        out_specs=[row(C2), row(C2), row(C2), row(C2)],
        out_shape=[jax.ShapeDtypeStruct((N, C2), F32)] * 4,
    )(accA, accB, s1, Wq, bq, Wk, bk, Wv, bv, Ws, bs)


def _final_body(acc_ref, s2_ref, out_ref):
    acc = acc_ref[0] + acc_ref[1]
    den = jnp.maximum(acc[:, C2:C2 + 1], 1e-16)
    h = acc[:, :C2] / den + s2_ref[...]
    avg = jnp.mean(h, axis=0, keepdims=True)
    mx = jnp.max(h, axis=0, keepdims=True)
    out_ref[...] = jnp.concatenate([avg, mx], axis=1)


def _final(acc2, s2):
    return pl.pallas_call(
        _final_body,
        out_shape=jax.ShapeDtypeStruct((1, 2 * C2), F32),
    )(acc2, s2)


_alpha1 = _make_alpha_kernel(C1)
_alpha2 = _make_alpha_kernel(C2)
_agg1a = _make_agg_kernel(D_IN, True)
_agg1b = _make_agg_kernel(D_IN, False)
_agg2 = _make_agg_kernel(C2, True)


def kernel(x, edge_index, edge_weight,
           Wq1, bq1, Wk1, bk1, Wv1, bv1, Ws1, bs1,
           Wq2, bq2, Wk2, bk2, Wv2, bv2, Ws2, bs2):
    src = edge_index[0]
    dst = edge_index[1]
    r1 = lambda b: b.reshape(1, -1)

    q1, k1, va, vb, s1 = _proj1(x, Wq1, r1(bq1), Wk1, r1(bk1),
                                Wv1, r1(bv1), Ws1, r1(bs1))
    alpha1, wmax1 = _alpha1(dst, src, q1, k1)
    accA = _agg1a(dst, src, alpha1, wmax1, va)
    accB = _agg1b(dst, src, alpha1, wmax1, vb)
    q2, k2, v2, s2 = _proj2(accA, accB, s1, Wq2, r1(bq2), Wk2, r1(bk2),
                            Wv2, r1(bv2), Ws2, r1(bs2))
    alpha2, wmax2 = _alpha2(dst, src, q2, k2)
    acc2 = _agg2(dst, src, alpha2, wmax2, v2)
    pooled = _final(acc2, s2)
    return pooled.reshape(2 * C2)


# speedup vs baseline: 6.8130x; 1.2576x over previous
import functools
import math

import jax
import jax.numpy as jnp
from jax import lax
from jax.experimental import pallas as pl
from jax.experimental.pallas import tpu as pltpu
from jax.experimental.pallas import tpu_sc as plsc

N = 10000
E = 320000
D_IN = 128
C1 = 256
C2 = 64

NC = 2
NS = 16
L = 16
NW = NC * NS
EPW = E // NW
CH = 80
NCH = EPW // CH

F32 = jnp.float32


def _mesh():
    return plsc.VectorSubcoreMesh(core_axis_name="c", subcore_axis_name="s")


def _make_alpha_kernel(C):
    NJ = C // L

    def body(dst_hbm, src_hbm, q_hbm, k_hbm, alpha_hbm, wmax_hbm,
             didx2, sidx2, qb2, kb2, pb, ab, mxb, sq0, sq1, sk0, sk1):
        cid = lax.axis_index("c")
        sid = lax.axis_index("s")
        wid = sid * NC + cid
        ebase = wid * EPW
        sq = (sq0, sq1)
        sk = (sk0, sk1)

        def issue(p, ch):
            off = pl.multiple_of(ebase + ch * CH, 8)
            pltpu.sync_copy(dst_hbm.at[pl.ds(off, CH)], didx2.at[p])
            pltpu.sync_copy(src_hbm.at[pl.ds(off, CH)], sidx2.at[p])
            pltpu.async_copy(q_hbm.at[didx2.at[p]], qb2.at[p], sq[p])
            pltpu.async_copy(k_hbm.at[sidx2.at[p]], kb2.at[p], sk[p])

        def compute(p, ch, mx):
            pltpu.make_async_copy(q_hbm.at[didx2.at[p]], qb2.at[p],
                                  sq[p]).wait()
            pltpu.make_async_copy(k_hbm.at[sidx2.at[p]], kb2.at[p],
                                  sk[p]).wait()

            def edge_body(e, carry):
                acc = qb2[p, e, pl.ds(0, L)] * kb2[p, e, pl.ds(0, L)]
                for j in range(1, NJ):
                    acc = acc + (qb2[p, e, pl.ds(j * L, L)] *
                                 kb2[p, e, pl.ds(j * L, L)])
                pb[e, pl.ds(0, L)] = acc
                return carry

            lax.fori_loop(0, CH, edge_body, 0, unroll=2)

            def group_body(g, mx_in):
                rows = g * L + lax.iota(jnp.int32, L)
                a16 = plsc.load_gather(pb, [rows, jnp.zeros((L,), jnp.int32)])
                for j in range(1, L):
                    a16 = a16 + plsc.load_gather(
                        pb, [rows, jnp.full((L,), j, jnp.int32)])
                ab[pl.ds(g * L, L)] = a16
                return jnp.maximum(mx_in, a16)

            mx = lax.fori_loop(0, CH // L, group_body, mx)
            off = pl.multiple_of(ebase + ch * CH, 8)
            pltpu.sync_copy(ab, alpha_hbm.at[pl.ds(off, CH)])
            return mx

        mx0 = jnp.full((L,), -jnp.inf, F32)
        issue(0, 0)

        def pair(t, mx):
            issue(1, 2 * t + 1)
            mx = compute(0, 2 * t, mx)

            @pl.when(2 * t + 2 < NCH)
            def _():
                issue(0, 2 * t + 2)
            mx = compute(1, 2 * t + 1, mx)
            return mx

        mx = lax.fori_loop(0, NCH // 2, pair, mx0)
        mx = compute(0, NCH - 1, mx)
        mxb[pl.ds(0, L)] = mx
        pltpu.sync_copy(mxb, wmax_hbm.at[wid])

    kern = pl.kernel(
        body,
        out_type=(jax.ShapeDtypeStruct((E,), F32),
                  jax.ShapeDtypeStruct((NW, L), F32)),
        mesh=_mesh(),
        compiler_params=pltpu.CompilerParams(needs_layout_passes=False,
                                             use_tc_tiling_on_sc=False),
        scratch_types=[
            pltpu.VMEM((2, CH), jnp.int32),
            pltpu.VMEM((2, CH), jnp.int32),
            pltpu.VMEM((2, CH, C), F32),
            pltpu.VMEM((2, CH, C), F32),
            pltpu.VMEM((CH, L), F32),
            pltpu.VMEM((CH,), F32),
            pltpu.VMEM((L,), F32),
            pltpu.SemaphoreType.DMA,
            pltpu.SemaphoreType.DMA,
            pltpu.SemaphoreType.DMA,
            pltpu.SemaphoreType.DMA,
        ],
    )
    return kern


def _make_agg_kernel(D, with_denom):
    CW = D + L if with_denom else D
    ZR = 8
    NBLK = N // ZR
    CHA = 80
    NCHA = EPW // CHA

    def body(dst_hbm, src_hbm, alpha_hbm, wmax_hbm, v_hbm, out_hbm,
             didx2, sidx2, vb2, sb, ab2, mb, zb, acc_sh, sv0, sv1):
        cid = lax.axis_index("c")
        sid = lax.axis_index("s")
        wid = sid * NC + cid
        ebase = wid * EPW
        sv = (sv0, sv1)

        def zfill(r, c):
            for j in range(CW // L):
                zb[r, pl.ds(j * L, L)] = jnp.zeros((L,), F32)
            return c
        lax.fori_loop(0, ZR, zfill, 0)

        def zcopy(t, c):
            blk = sid + t * NS

            @pl.when(blk < NBLK)
            def _():
                off = pl.multiple_of(blk * ZR, 8)
                pltpu.sync_copy(zb, acc_sh.at[pl.ds(off, ZR)])
            return c
        lax.fori_loop(0, (NBLK + NS - 1) // NS, zcopy, 0)

        pltpu.sync_copy(wmax_hbm, mb)

        def mred(i, m):
            return jnp.maximum(m, mb[i, pl.ds(0, L)])
        m16 = lax.fori_loop(0, NW, mred, jnp.full((L,), -jnp.inf, F32))
        gmax = jnp.max(m16)

        plsc.subcore_barrier()

        def issue(p, ch):
            off = pl.multiple_of(ebase + ch * CHA, 8)
            pltpu.sync_copy(dst_hbm.at[pl.ds(off, CHA)], didx2.at[p])
            pltpu.sync_copy(src_hbm.at[pl.ds(off, CHA)], sidx2.at[p])
            pltpu.sync_copy(alpha_hbm.at[pl.ds(off, CHA)], ab2.at[p])
            pltpu.async_copy(v_hbm.at[sidx2.at[p]], vb2.at[p], sv[p])

        def compute(p, ch):
            pltpu.make_async_copy(v_hbm.at[sidx2.at[p]], vb2.at[p],
                                  sv[p]).wait()

            def group_body(g, c):
                a16 = ab2[p, pl.ds(g * L, L)]
                e16 = jnp.exp(a16 - gmax)
                for l in range(L):
                    s = e16[l]
                    e = g * L + l
                    for j in range(D // L):
                        sb[e, pl.ds(j * L, L)] = (vb2[p, e, pl.ds(j * L, L)]
                                                  * s)
                    if with_denom:
                        sb[e, pl.ds(D, L)] = jnp.broadcast_to(s, (L,))
                return c
            lax.fori_loop(0, CHA // L, group_body, 0)

            pltpu.sync_copy(sb, acc_sh.at[didx2.at[p]], add=True)

        issue(0, 0)

        def pair(t, c):
            issue(1, 2 * t + 1)
            compute(0, 2 * t)

            @pl.when(2 * t + 2 < NCHA)
            def _():
                issue(0, 2 * t + 2)
            compute(1, 2 * t + 1)
            return c

        lax.fori_loop(0, NCHA // 2, pair, 0)
        if NCHA % 2 == 1:
            compute(0, NCHA - 1)

        plsc.subcore_barrier()

        @pl.when(sid == 0)
        def _():
            pltpu.sync_copy(acc_sh, out_hbm.at[cid])

    kern = pl.kernel(
        body,
        out_type=jax.ShapeDtypeStruct((NC, N, CW), F32),
        mesh=_mesh(),
        compiler_params=pltpu.CompilerParams(needs_layout_passes=False,
                                             use_tc_tiling_on_sc=False),
        scratch_types=[
            pltpu.VMEM((2, CHA), jnp.int32),
            pltpu.VMEM((2, CHA), jnp.int32),
            pltpu.VMEM((2, CHA, D), F32),
            pltpu.VMEM((CHA, CW), F32),
            pltpu.VMEM((2, CHA), F32),
            pltpu.VMEM((NW, L), F32),
            pltpu.VMEM((ZR, CW), F32),
            pltpu.VMEM_SHARED((N, CW), F32),
            pltpu.SemaphoreType.DMA,
            pltpu.SemaphoreType.DMA,
        ],
    )
    return kern


_RB = 1000


def _proj1_body(x_ref, wq, bq, wk, bk, wv, bv, ws, bs,
                q_ref, k_ref, va_ref, vb_ref, s_ref):
    xb = jnp.nan_to_num(x_ref[...], nan=0.0)
    scale = 1.0 / math.sqrt(C1)
    q = (jnp.dot(xb, wq[...], preferred_element_type=F32) + bq[...]) * scale
    k = jnp.dot(xb, wk[...], preferred_element_type=F32) + bk[...]
    v = jnp.dot(xb, wv[...], preferred_element_type=F32) + bv[...]
    s = jnp.dot(xb, ws[...], preferred_element_type=F32) + bs[...]
    q_ref[...] = q
    k_ref[...] = k
    va_ref[...] = v[:, :D_IN]
    vb_ref[...] = v[:, D_IN:]
    s_ref[...] = s


def _proj1(x, Wq, bq, Wk, bk, Wv, bv, Ws, bs):
    grid = N // _RB
    full = lambda r, c: pl.BlockSpec((r, c), lambda i: (0, 0))
    row = lambda c: pl.BlockSpec((_RB, c), lambda i: (i, 0))
    return pl.pallas_call(
        _proj1_body,
        grid=(grid,),
        in_specs=[row(D_IN),
                  full(D_IN, C1), full(1, C1), full(D_IN, C1), full(1, C1),
                  full(D_IN, C1), full(1, C1), full(D_IN, C1), full(1, C1)],
        out_specs=[row(C1), row(C1), row(D_IN), row(D_IN), row(C1)],
        out_shape=[jax.ShapeDtypeStruct((N, C1), F32),
                   jax.ShapeDtypeStruct((N, C1), F32),
                   jax.ShapeDtypeStruct((N, D_IN), F32),
                   jax.ShapeDtypeStruct((N, D_IN), F32),
                   jax.ShapeDtypeStruct((N, C1), F32)],
    )(x, Wq, bq, Wk, bk, Wv, bv, Ws, bs)


def _proj2_body(aa_ref, ab_ref, s1_ref, wq, bq, wk, bk, wv, bv, ws, bs,
                q_ref, k_ref, v_ref, s_ref):
    accA = aa_ref[0] + aa_ref[1]
    accB = ab_ref[0] + ab_ref[1]
    den = jnp.maximum(accA[:, D_IN:D_IN + 1], 1e-16)
    h = jnp.concatenate([accA[:, :D_IN], accB], axis=1) / den + s1_ref[...]
    scale = 1.0 / math.sqrt(C2)
    q = (jnp.dot(h, wq[...], preferred_element_type=F32) + bq[...]) * scale
    k = jnp.dot(h, wk[...], preferred_element_type=F32) + bk[...]
    v = jnp.dot(h, wv[...], preferred_element_type=F32) + bv[...]
    s = jnp.dot(h, ws[...], preferred_element_type=F32) + bs[...]
    q_ref[...] = q
    k_ref[...] = k
    v_ref[...] = v
    s_ref[...] = s


def _proj2(accA, accB, s1, Wq, bq, Wk, bk, Wv, bv, Ws, bs):
    grid = N // _RB
    full = lambda r, c: pl.BlockSpec((r, c), lambda i: (0, 0))
    row = lambda c: pl.BlockSpec((_RB, c), lambda i: (i, 0))
    acc = lambda c: pl.BlockSpec((NC, _RB, c), lambda i: (0, i, 0))
    return pl.pallas_call(
        _proj2_body,
        grid=(grid,),
        in_specs=[acc(D_IN + L), acc(D_IN), row(C1),
                  full(C1, C2), full(1, C2), full(C1, C2), full(1, C2),
                  full(C1, C2), full(1, C2), full(C1, C2), full(1, C2)],
        out_specs=[row(C2), row(C2), row(C2), row(C2)],
        out_shape=[jax.ShapeDtypeStruct((N, C2), F32)] * 4,
    )(accA, accB, s1, Wq, bq, Wk, bk, Wv, bv, Ws, bs)


def _final_body(acc_ref, s2_ref, out_ref):
    acc = acc_ref[0] + acc_ref[1]
    den = jnp.maximum(acc[:, C2:C2 + 1], 1e-16)
    h = acc[:, :C2] / den + s2_ref[...]
    avg = jnp.mean(h, axis=0, keepdims=True)
    mx = jnp.max(h, axis=0, keepdims=True)
    out_ref[...] = jnp.concatenate([avg, mx], axis=1)


def _final(acc2, s2):
    return pl.pallas_call(
        _final_body,
        out_shape=jax.ShapeDtypeStruct((1, 2 * C2), F32),
    )(acc2, s2)


_alpha1 = _make_alpha_kernel(C1)
_alpha2 = _make_alpha_kernel(C2)
_agg1a = _make_agg_kernel(D_IN, True)
_agg1b = _make_agg_kernel(D_IN, False)
_agg2 = _make_agg_kernel(C2, True)


def kernel(x, edge_index, edge_weight,
           Wq1, bq1, Wk1, bk1, Wv1, bv1, Ws1, bs1,
           Wq2, bq2, Wk2, bk2, Wv2, bv2, Ws2, bs2):
    src = edge_index[0]
    dst = edge_index[1]
    r1 = lambda b: b.reshape(1, -1)

    q1, k1, va, vb, s1 = _proj1(x, Wq1, r1(bq1), Wk1, r1(bk1),
                                Wv1, r1(bv1), Ws1, r1(bs1))
    alpha1, wmax1 = _alpha1(dst, src, q1, k1)
    accA = _agg1a(dst, src, alpha1, wmax1, va)
    accB = _agg1b(dst, src, alpha1, wmax1, vb)
    q2, k2, v2, s2 = _proj2(accA, accB, s1, Wq2, r1(bq2), Wk2, r1(bk2),
                            Wv2, r1(bv2), Ws2, r1(bs2))
    alpha2, wmax2 = _alpha2(dst, src, q2, k2)
    acc2 = _agg2(dst, src, alpha2, wmax2, v2)
    pooled = _final(acc2, s2)
    return pooled.reshape(2 * C2)


# speedup vs baseline: 9.3849x; 1.3775x over previous
import functools
import math

import jax
import jax.numpy as jnp
from jax import lax
from jax.experimental import pallas as pl
from jax.experimental.pallas import tpu as pltpu
from jax.experimental.pallas import tpu_sc as plsc

N = 10000
E = 320000
D_IN = 128
C1 = 256
C2 = 64

NC = 2
NS = 16
L = 16
NW = NC * NS
EPW = E // NW
CH = 80
NCH = EPW // CH

F32 = jnp.float32


def _mesh():
    return plsc.VectorSubcoreMesh(core_axis_name="c", subcore_axis_name="s")


def _make_alpha_kernel(C):
    NJ = C // L

    def body(dst3, src3, q_hbm, k_hbm, alpha_out, wmax_hbm,
             didx_all, sidx_all, qb2, kb2, pb, ab_all, mxb,
             sq0, sq1, sk0, sk1):
        cid = lax.axis_index("c")
        sid = lax.axis_index("s")
        wid = sid * NC + cid
        sq = (sq0, sq1)
        sk = (sk0, sk1)

        pltpu.sync_copy(dst3.at[wid], didx_all)
        pltpu.sync_copy(src3.at[wid], sidx_all)

        def issue(p, ch):
            pltpu.async_copy(q_hbm.at[didx_all.at[ch]], qb2.at[p], sq[p])
            pltpu.async_copy(k_hbm.at[sidx_all.at[ch]], kb2.at[p], sk[p])

        def compute(p, ch, mx):
            pltpu.make_async_copy(q_hbm.at[didx_all.at[ch]], qb2.at[p],
                                  sq[p]).wait()
            pltpu.make_async_copy(k_hbm.at[sidx_all.at[ch]], kb2.at[p],
                                  sk[p]).wait()

            def edge_body(e, carry):
                acc = qb2[p, e, pl.ds(0, L)] * kb2[p, e, pl.ds(0, L)]
                for j in range(1, NJ):
                    acc = acc + (qb2[p, e, pl.ds(j * L, L)] *
                                 kb2[p, e, pl.ds(j * L, L)])
                pb[e, pl.ds(0, L)] = acc
                return carry

            lax.fori_loop(0, CH, edge_body, 0, unroll=2)

            def group_body(g, mx_in):
                rows = g * L + lax.iota(jnp.int32, L)
                a16 = plsc.load_gather(pb, [rows, jnp.zeros((L,), jnp.int32)])
                for j in range(1, L):
                    a16 = a16 + plsc.load_gather(
                        pb, [rows, jnp.full((L,), j, jnp.int32)])
                ab_all[ch, pl.ds(g * L, L)] = a16
                return jnp.maximum(mx_in, a16)

            mx = lax.fori_loop(0, CH // L, group_body, mx)
            return mx

        mx0 = jnp.full((L,), -jnp.inf, F32)
        issue(0, 0)

        def pair(t, mx):
            issue(1, 2 * t + 1)
            mx = compute(0, 2 * t, mx)
            issue(0, 2 * t + 2)
            mx = compute(1, 2 * t + 1, mx)
            return mx

        mx = lax.fori_loop(0, NCH // 2, pair, mx0)
        mx = compute(0, NCH - 1, mx)
        pltpu.sync_copy(ab_all, alpha_out.at[wid])
        mxb[pl.ds(0, L)] = mx
        pltpu.sync_copy(mxb, wmax_hbm.at[wid])

    kern = pl.kernel(
        body,
        out_type=(jax.ShapeDtypeStruct((NW, NCH, CH), F32),
                  jax.ShapeDtypeStruct((NW, L), F32)),
        mesh=_mesh(),
        compiler_params=pltpu.CompilerParams(needs_layout_passes=False,
                                             use_tc_tiling_on_sc=False),
        scratch_types=[
            pltpu.VMEM((NCH, CH), jnp.int32),
            pltpu.VMEM((NCH, CH), jnp.int32),
            pltpu.VMEM((2, CH, C), F32),
            pltpu.VMEM((2, CH, C), F32),
            pltpu.VMEM((CH, L), F32),
            pltpu.VMEM((NCH, CH), F32),
            pltpu.VMEM((L,), F32),
            pltpu.SemaphoreType.DMA,
            pltpu.SemaphoreType.DMA,
            pltpu.SemaphoreType.DMA,
            pltpu.SemaphoreType.DMA,
        ],
    )
    return kern


def _make_agg_kernel(D, with_denom, cps, den_test=False):
    CW = D + L if with_denom else D
    ZR = 8
    NBLK = N // ZR
    CHA = 80
    NCHA = EPW // CHA
    NSLAB = NCHA // cps

    def body(dst3, src3, alpha3, wmax_hbm, v_hbm, *refs):
        if den_test:
            (out_hbm, den_hbm, didx_s, sidx_s, vb2, sb, ab_s, mb, zb,
             denb, acc_sh, sv0, sv1) = refs
        else:
            (out_hbm, didx_s, sidx_s, vb2, sb, ab_s, mb, zb,
             acc_sh, sv0, sv1) = refs
        cid = lax.axis_index("c")
        sid = lax.axis_index("s")
        wid = sid * NC + cid
        sv = (sv0, sv1)

        def zfill(r, c):
            for j in range(CW // L):
                zb[r, pl.ds(j * L, L)] = jnp.zeros((L,), F32)
            return c
        lax.fori_loop(0, ZR, zfill, 0)

        def zcopy(t, c):
            blk = sid + t * NS

            @pl.when(blk < NBLK)
            def _():
                off = pl.multiple_of(blk * ZR, 8)
                pltpu.sync_copy(zb, acc_sh.at[pl.ds(off, ZR)])
            return c
        lax.fori_loop(0, (NBLK + NS - 1) // NS, zcopy, 0)

        pltpu.sync_copy(wmax_hbm, mb)

        def mred(i, m):
            return jnp.maximum(m, mb[i, pl.ds(0, L)])
        m16 = lax.fori_loop(0, NW, mred, jnp.full((L,), -jnp.inf, F32))
        gmax = jnp.max(m16)

        if den_test:
            def dzero(i, c):
                denb[pl.ds(i * L, L)] = jnp.zeros((L,), F32)
                return c
            lax.fori_loop(0, N // L, dzero, 0)

        plsc.subcore_barrier()

        def issue(p, j):
            pltpu.async_copy(v_hbm.at[sidx_s.at[j]], vb2.at[p], sv[p])

        def compute(p, j):
            pltpu.make_async_copy(v_hbm.at[sidx_s.at[j]], vb2.at[p],
                                  sv[p]).wait()

            def group_body(g, c):
                a16 = ab_s[j, pl.ds(g * L, L)]
                e16 = jnp.exp(a16 - gmax)
                if den_test:
                    d16 = didx_s[j, pl.ds(g * L, L)]
                    plsc.addupdate_scatter(denb, [d16], e16)
                for l in range(L):
                    s = e16[l]
                    e = g * L + l
                    for jj in range(D // L):
                        sb[e, pl.ds(jj * L, L)] = (vb2[p, e, pl.ds(jj * L, L)]
                                                   * s)
                    if with_denom:
                        sb[e, pl.ds(D, L)] = jnp.broadcast_to(s, (L,))
                return c
            lax.fori_loop(0, CHA // L, group_body, 0)

            pltpu.sync_copy(sb, acc_sh.at[didx_s.at[j]], add=True)

        def slab(si, c):
            pltpu.sync_copy(dst3.at[wid, pl.ds(si * cps, cps)], didx_s)
            pltpu.sync_copy(src3.at[wid, pl.ds(si * cps, cps)], sidx_s)
            pltpu.sync_copy(alpha3.at[wid, pl.ds(si * cps, cps)], ab_s)
            issue(0, 0)

            def pair(t, cc):
                issue(1, 2 * t + 1)
                compute(0, 2 * t)
                issue(0, 2 * t + 2)
                compute(1, 2 * t + 1)
                return cc
            lax.fori_loop(0, cps // 2, pair, 0)
            compute(0, cps - 1)
            return c

        lax.fori_loop(0, NSLAB, slab, 0)

        plsc.subcore_barrier()

        @pl.when(sid == 0)
        def _():
            pltpu.sync_copy(acc_sh, out_hbm.at[cid])
        if den_test:
            pltpu.sync_copy(denb, den_hbm.at[wid])

    out_type = [jax.ShapeDtypeStruct((NC, N, CW), F32)]
    if den_test:
        out_type.append(jax.ShapeDtypeStruct((NW, N), F32))
    scratch = [
        pltpu.VMEM((cps, CHA), jnp.int32),
        pltpu.VMEM((cps, CHA), jnp.int32),
        pltpu.VMEM((2, CHA, D), F32),
        pltpu.VMEM((CHA, CW), F32),
        pltpu.VMEM((cps, CHA), F32),
        pltpu.VMEM((NW, L), F32),
        pltpu.VMEM((ZR, CW), F32),
    ]
    if den_test:
        scratch.append(pltpu.VMEM((N,), F32))
    scratch += [
        pltpu.VMEM_SHARED((N, CW), F32),
        pltpu.SemaphoreType.DMA,
        pltpu.SemaphoreType.DMA,
    ]
    kern = pl.kernel(
        body,
        out_type=tuple(out_type) if den_test else out_type[0],
        mesh=_mesh(),
        compiler_params=pltpu.CompilerParams(needs_layout_passes=False,
                                             use_tc_tiling_on_sc=False),
        scratch_types=scratch,
    )
    return kern


_RB = 1000


def _proj1_body(x_ref, wq, bq, wk, bk, wv, bv, ws, bs,
                q_ref, k_ref, va_ref, vb_ref, s_ref):
    xb = jnp.nan_to_num(x_ref[...], nan=0.0)
    scale = 1.0 / math.sqrt(C1)
    q = (jnp.dot(xb, wq[...], preferred_element_type=F32) + bq[...]) * scale
    k = jnp.dot(xb, wk[...], preferred_element_type=F32) + bk[...]
    v = jnp.dot(xb, wv[...], preferred_element_type=F32) + bv[...]
    s = jnp.dot(xb, ws[...], preferred_element_type=F32) + bs[...]
    q_ref[...] = q
    k_ref[...] = k
    va_ref[...] = v[:, :D_IN]
    vb_ref[...] = v[:, D_IN:]
    s_ref[...] = s


def _proj1(x, Wq, bq, Wk, bk, Wv, bv, Ws, bs):
    grid = N // _RB
    full = lambda r, c: pl.BlockSpec((r, c), lambda i: (0, 0))
    row = lambda c: pl.BlockSpec((_RB, c), lambda i: (i, 0))
    return pl.pallas_call(
        _proj1_body,
        grid=(grid,),
        in_specs=[row(D_IN),
                  full(D_IN, C1), full(1, C1), full(D_IN, C1), full(1, C1),
                  full(D_IN, C1), full(1, C1), full(D_IN, C1), full(1, C1)],
        out_specs=[row(C1), row(C1), row(D_IN), row(D_IN), row(C1)],
        out_shape=[jax.ShapeDtypeStruct((N, C1), F32),
                   jax.ShapeDtypeStruct((N, C1), F32),
                   jax.ShapeDtypeStruct((N, D_IN), F32),
                   jax.ShapeDtypeStruct((N, D_IN), F32),
                   jax.ShapeDtypeStruct((N, C1), F32)],
    )(x, Wq, bq, Wk, bk, Wv, bv, Ws, bs)


def _proj2_body(aa_ref, ab_ref, s1_ref, wq, bq, wk, bk, wv, bv, ws, bs,
                q_ref, k_ref, v_ref, s_ref):
    accA = aa_ref[0] + aa_ref[1]
    accB = ab_ref[0] + ab_ref[1]
    den = jnp.maximum(accA[:, D_IN:D_IN + 1], 1e-16)
    h = jnp.concatenate([accA[:, :D_IN], accB], axis=1) / den + s1_ref[...]
    scale = 1.0 / math.sqrt(C2)
    q = (jnp.dot(h, wq[...], preferred_element_type=F32) + bq[...]) * scale
    k = jnp.dot(h, wk[...], preferred_element_type=F32) + bk[...]
    v = jnp.dot(h, wv[...], preferred_element_type=F32) + bv[...]
    s = jnp.dot(h, ws[...], preferred_element_type=F32) + bs[...]
    q_ref[...] = q
    k_ref[...] = k
    v_ref[...] = v
    s_ref[...] = s


def _proj2(accA, accB, s1, Wq, bq, Wk, bk, Wv, bv, Ws, bs):
    grid = N // _RB
    full = lambda r, c: pl.BlockSpec((r, c), lambda i: (0, 0))
    row = lambda c: pl.BlockSpec((_RB, c), lambda i: (i, 0))
    acc = lambda c: pl.BlockSpec((NC, _RB, c), lambda i: (0, i, 0))
    return pl.pallas_call(
        _proj2_body,
        grid=(grid,),
        in_specs=[acc(D_IN + L), acc(D_IN), row(C1),
                  full(C1, C2), full(1, C2), full(C1, C2), full(1, C2),
                  full(C1, C2), full(1, C2), full(C1, C2), full(1, C2)],
        out_specs=[row(C2), row(C2), row(C2), row(C2)],
        out_shape=[jax.ShapeDtypeStruct((N, C2), F32)] * 4,
    )(accA, accB, s1, Wq, bq, Wk, bk, Wv, bv, Ws, bs)


def _final_body(acc_ref, s2_ref, den_ref, out_ref):
    acc = acc_ref[0] + acc_ref[1]
    den = jnp.maximum(acc[:, C2:C2 + 1], 1e-16)
    h = acc[:, :C2] / den + s2_ref[...]
    dn = jnp.sum(den_ref[...], axis=0)
    h = h + 30.0 * (dn - acc[:, C2])[:, None]
    avg = jnp.mean(h, axis=0, keepdims=True)
    mx = jnp.max(h, axis=0, keepdims=True)
    out_ref[...] = jnp.concatenate([avg, mx], axis=1)


def _final(acc2, s2, den2):
    return pl.pallas_call(
        _final_body,
        out_shape=jax.ShapeDtypeStruct((1, 2 * C2), F32),
    )(acc2, s2, den2)


_alpha1 = _make_alpha_kernel(C1)
_alpha2 = _make_alpha_kernel(C2)
_agg1a = _make_agg_kernel(D_IN, True, 5)
_agg1b = _make_agg_kernel(D_IN, False, 25)
_agg2 = _make_agg_kernel(C2, True, 25, den_test=True)


def kernel(x, edge_index, edge_weight,
           Wq1, bq1, Wk1, bk1, Wv1, bv1, Ws1, bs1,
           Wq2, bq2, Wk2, bk2, Wv2, bv2, Ws2, bs2):
    src3 = edge_index[0].reshape(NW, NCH, CH)
    dst3 = edge_index[1].reshape(NW, NCH, CH)
    r1 = lambda b: b.reshape(1, -1)

    q1, k1, va, vb, s1 = _proj1(x, Wq1, r1(bq1), Wk1, r1(bk1),
                                Wv1, r1(bv1), Ws1, r1(bs1))
    alpha1, wmax1 = _alpha1(dst3, src3, q1, k1)
    accA = _agg1a(dst3, src3, alpha1, wmax1, va)
    accB = _agg1b(dst3, src3, alpha1, wmax1, vb)
    q2, k2, v2, s2 = _proj2(accA, accB, s1, Wq2, r1(bq2), Wk2, r1(bk2),
                            Wv2, r1(bv2), Ws2, r1(bs2))
    alpha2, wmax2 = _alpha2(dst3, src3, q2, k2)
    acc2, den2 = _agg2(dst3, src3, alpha2, wmax2, v2)
    pooled = _final(acc2, s2, den2)
    return pooled.reshape(2 * C2)


# speedup vs baseline: 13.0119x; 1.3865x over previous
import functools
import math

import jax
import jax.numpy as jnp
from jax import lax
from jax.experimental import pallas as pl
from jax.experimental.pallas import tpu as pltpu
from jax.experimental.pallas import tpu_sc as plsc

N = 10000
E = 320000
D_IN = 128
C1 = 256
C2 = 64

NC = 2
NS = 16
L = 16
NW = NC * NS
EPW = E // NW
CH = 80
NCH = EPW // CH

F32 = jnp.float32


def _mesh():
    return plsc.VectorSubcoreMesh(core_axis_name="c", subcore_axis_name="s")


def _make_alpha_kernel(C):
    NJ = C // L

    def body(dst3, src3, q_hbm, k_hbm, alpha_out, wmax_hbm,
             didx_all, sidx_all, qb2, kb2, pb, ab_all, mxb,
             sq0, sq1, sk0, sk1):
        cid = lax.axis_index("c")
        sid = lax.axis_index("s")
        wid = sid * NC + cid
        sq = (sq0, sq1)
        sk = (sk0, sk1)

        pltpu.sync_copy(dst3.at[wid], didx_all)
        pltpu.sync_copy(src3.at[wid], sidx_all)

        def issue(p, ch):
            pltpu.async_copy(q_hbm.at[didx_all.at[ch]], qb2.at[p], sq[p])
            pltpu.async_copy(k_hbm.at[sidx_all.at[ch]], kb2.at[p], sk[p])

        def compute(p, ch, mx):
            pltpu.make_async_copy(q_hbm.at[didx_all.at[ch]], qb2.at[p],
                                  sq[p]).wait()
            pltpu.make_async_copy(k_hbm.at[sidx_all.at[ch]], kb2.at[p],
                                  sk[p]).wait()

            def edge_body(e, carry):
                acc = qb2[p, e, pl.ds(0, L)] * kb2[p, e, pl.ds(0, L)]
                for j in range(1, NJ):
                    acc = acc + (qb2[p, e, pl.ds(j * L, L)] *
                                 kb2[p, e, pl.ds(j * L, L)])
                pb[e, pl.ds(0, L)] = acc
                return carry

            lax.fori_loop(0, CH, edge_body, 0, unroll=2)

            def group_body(g, mx_in):
                rows = g * L + lax.iota(jnp.int32, L)
                a16 = plsc.load_gather(pb, [rows, jnp.zeros((L,), jnp.int32)])
                for j in range(1, L):
                    a16 = a16 + plsc.load_gather(
                        pb, [rows, jnp.full((L,), j, jnp.int32)])
                ab_all[ch, pl.ds(g * L, L)] = a16
                return jnp.maximum(mx_in, a16)

            mx = lax.fori_loop(0, CH // L, group_body, mx)
            return mx

        mx0 = jnp.full((L,), -jnp.inf, F32)
        issue(0, 0)

        def pair(t, mx):
            issue(1, 2 * t + 1)
            mx = compute(0, 2 * t, mx)
            issue(0, 2 * t + 2)
            mx = compute(1, 2 * t + 1, mx)
            return mx

        mx = lax.fori_loop(0, NCH // 2, pair, mx0)
        mx = compute(0, NCH - 1, mx)
        pltpu.sync_copy(ab_all, alpha_out.at[wid])
        mxb[pl.ds(0, L)] = mx
        pltpu.sync_copy(mxb, wmax_hbm.at[wid])

    kern = pl.kernel(
        body,
        out_type=(jax.ShapeDtypeStruct((NW, NCH, CH), F32),
                  jax.ShapeDtypeStruct((NW, L), F32)),
        mesh=_mesh(),
        compiler_params=pltpu.CompilerParams(needs_layout_passes=False,
                                             use_tc_tiling_on_sc=False),
        scratch_types=[
            pltpu.VMEM((NCH, CH), jnp.int32),
            pltpu.VMEM((NCH, CH), jnp.int32),
            pltpu.VMEM((2, CH, C), F32),
            pltpu.VMEM((2, CH, C), F32),
            pltpu.VMEM((CH, L), F32),
            pltpu.VMEM((NCH, CH), F32),
            pltpu.VMEM((L,), F32),
            pltpu.SemaphoreType.DMA,
            pltpu.SemaphoreType.DMA,
            pltpu.SemaphoreType.DMA,
            pltpu.SemaphoreType.DMA,
        ],
    )
    return kern


def _make_agg_kernel(D, with_den, cps):
    ZR = 8
    NBLK = N // ZR
    CHA = 80
    NCHA = EPW // CHA
    NSLAB = NCHA // cps

    def body(dst3, src3, alpha3, wmax_hbm, v_hbm, *refs):
        if with_den:
            (out_hbm, den_hbm, didx_s, sidx_s, vb2, ab_s, mb, zb,
             denb, acc_sh, sv0, sv1) = refs
        else:
            (out_hbm, didx_s, sidx_s, vb2, ab_s, mb, zb,
             acc_sh, sv0, sv1) = refs
        cid = lax.axis_index("c")
        sid = lax.axis_index("s")
        wid = sid * NC + cid
        sv = (sv0, sv1)

        def zfill(r, c):
            for j in range(D // L):
                zb[r, pl.ds(j * L, L)] = jnp.zeros((L,), F32)
            return c
        lax.fori_loop(0, ZR, zfill, 0)

        def zcopy(t, c):
            blk = sid + t * NS

            @pl.when(blk < NBLK)
            def _():
                off = pl.multiple_of(blk * ZR, 8)
                pltpu.sync_copy(zb, acc_sh.at[pl.ds(off, ZR)])
            return c
        lax.fori_loop(0, (NBLK + NS - 1) // NS, zcopy, 0)

        pltpu.sync_copy(wmax_hbm, mb)

        def mred(i, m):
            return jnp.maximum(m, mb[i, pl.ds(0, L)])
        m16 = lax.fori_loop(0, NW, mred, jnp.full((L,), -jnp.inf, F32))
        gmax = jnp.max(m16)

        if with_den:
            def dzero(i, c):
                denb[pl.ds(i * L, L)] = jnp.zeros((L,), F32)
                return c
            lax.fori_loop(0, N // L, dzero, 0)

        plsc.subcore_barrier()

        def issue(p, j):
            pltpu.async_copy(v_hbm.at[sidx_s.at[j]], vb2.at[p], sv[p])

        def compute(p, j):
            pltpu.make_async_copy(v_hbm.at[sidx_s.at[j]], vb2.at[p],
                                  sv[p]).wait()

            def group_body(g, c):
                a16 = ab_s[j, pl.ds(g * L, L)]
                e16 = jnp.exp(a16 - gmax)
                if with_den:
                    d16 = didx_s[j, pl.ds(g * L, L)]
                    plsc.addupdate_scatter(denb, [d16], e16)
                for l in range(L):
                    s = e16[l]
                    e = g * L + l
                    for jj in range(D // L):
                        vb2[p, e, pl.ds(jj * L, L)] = (
                            vb2[p, e, pl.ds(jj * L, L)] * s)
                return c
            lax.fori_loop(0, CHA // L, group_body, 0)

            pltpu.sync_copy(vb2.at[p], acc_sh.at[didx_s.at[j]], add=True)

        def slab(si, c):
            pltpu.sync_copy(dst3.at[wid, pl.ds(si * cps, cps)], didx_s)
            pltpu.sync_copy(src3.at[wid, pl.ds(si * cps, cps)], sidx_s)
            pltpu.sync_copy(alpha3.at[wid, pl.ds(si * cps, cps)], ab_s)
            issue(0, 0)

            def pair(t, cc):
                issue(1, 2 * t + 1)
                compute(0, 2 * t)
                issue(0, 2 * t + 2)
                compute(1, 2 * t + 1)
                return cc
            lax.fori_loop(0, cps // 2, pair, 0)
            compute(0, cps - 1)
            return c

        lax.fori_loop(0, NSLAB, slab, 0)

        plsc.subcore_barrier()

        @pl.when(sid == 0)
        def _():
            pltpu.sync_copy(acc_sh, out_hbm.at[cid])
        if with_den:
            pltpu.sync_copy(denb, den_hbm.at[wid])

    out_type = [jax.ShapeDtypeStruct((NC, N, D), F32)]
    if with_den:
        out_type.append(jax.ShapeDtypeStruct((NW, N), F32))
    scratch = [
        pltpu.VMEM((cps, CHA), jnp.int32),
        pltpu.VMEM((cps, CHA), jnp.int32),
        pltpu.VMEM((2, CHA, D), F32),
        pltpu.VMEM((cps, CHA), F32),
        pltpu.VMEM((NW, L), F32),
        pltpu.VMEM((ZR, D), F32),
    ]
    if with_den:
        scratch.append(pltpu.VMEM((N,), F32))
    scratch += [
        pltpu.VMEM_SHARED((N, D), F32),
        pltpu.SemaphoreType.DMA,
        pltpu.SemaphoreType.DMA,
    ]
    kern = pl.kernel(
        body,
        out_type=tuple(out_type) if with_den else out_type[0],
        mesh=_mesh(),
        compiler_params=pltpu.CompilerParams(needs_layout_passes=False,
                                             use_tc_tiling_on_sc=False),
        scratch_types=scratch,
    )
    return kern


_RB = 1000


def _proj1_body(x_ref, wq, bq, wk, bk, wv, bv, ws, bs,
                q_ref, k_ref, va_ref, vb_ref, s_ref):
    xb = jnp.nan_to_num(x_ref[...], nan=0.0)
    scale = 1.0 / math.sqrt(C1)
    q = (jnp.dot(xb, wq[...], preferred_element_type=F32) + bq[...]) * scale
    k = jnp.dot(xb, wk[...], preferred_element_type=F32) + bk[...]
    v = jnp.dot(xb, wv[...], preferred_element_type=F32) + bv[...]
    s = jnp.dot(xb, ws[...], preferred_element_type=F32) + bs[...]
    q_ref[...] = q
    k_ref[...] = k
    va_ref[...] = v[:, :D_IN]
    vb_ref[...] = v[:, D_IN:]
    s_ref[...] = s


def _proj1(x, Wq, bq, Wk, bk, Wv, bv, Ws, bs):
    grid = N // _RB
    full = lambda r, c: pl.BlockSpec((r, c), lambda i: (0, 0))
    row = lambda c: pl.BlockSpec((_RB, c), lambda i: (i, 0))
    return pl.pallas_call(
        _proj1_body,
        grid=(grid,),
        in_specs=[row(D_IN),
                  full(D_IN, C1), full(1, C1), full(D_IN, C1), full(1, C1),
                  full(D_IN, C1), full(1, C1), full(D_IN, C1), full(1, C1)],
        out_specs=[row(C1), row(C1), row(D_IN), row(D_IN), row(C1)],
        out_shape=[jax.ShapeDtypeStruct((N, C1), F32),
                   jax.ShapeDtypeStruct((N, C1), F32),
                   jax.ShapeDtypeStruct((N, D_IN), F32),
                   jax.ShapeDtypeStruct((N, D_IN), F32),
                   jax.ShapeDtypeStruct((N, C1), F32)],
    )(x, Wq, bq, Wk, bk, Wv, bv, Ws, bs)


def _proj2_body(aa_ref, ab_ref, den_ref, s1_ref, wq, bq, wk, bk, wv, bv,
                ws, bs, q_ref, k_ref, v_ref, s_ref):
    accA = aa_ref[0] + aa_ref[1]
    accB = ab_ref[0] + ab_ref[1]
    dn = jnp.sum(den_ref[...], axis=1)
    den = jnp.maximum(dn, 1e-16)[:, None]
    h = jnp.concatenate([accA, accB], axis=1) / den + s1_ref[...]
    scale = 1.0 / math.sqrt(C2)
    q = (jnp.dot(h, wq[...], preferred_element_type=F32) + bq[...]) * scale
    k = jnp.dot(h, wk[...], preferred_element_type=F32) + bk[...]
    v = jnp.dot(h, wv[...], preferred_element_type=F32) + bv[...]
    s = jnp.dot(h, ws[...], preferred_element_type=F32) + bs[...]
    q_ref[...] = q
    k_ref[...] = k
    v_ref[...] = v
    s_ref[...] = s


def _proj2(accA, accB, den1, s1, Wq, bq, Wk, bk, Wv, bv, Ws, bs):
    grid = N // _RB
    full = lambda r, c: pl.BlockSpec((r, c), lambda i: (0, 0))
    row = lambda c: pl.BlockSpec((_RB, c), lambda i: (i, 0))
    acc = lambda c: pl.BlockSpec((NC, _RB, c), lambda i: (0, i, 0))
    dspec = pl.BlockSpec((_RB, NW), lambda i: (i, 0))
    return pl.pallas_call(
        _proj2_body,
        grid=(grid,),
        in_specs=[acc(D_IN), acc(D_IN), dspec, row(C1),
                  full(C1, C2), full(1, C2), full(C1, C2), full(1, C2),
                  full(C1, C2), full(1, C2), full(C1, C2), full(1, C2)],
        out_specs=[row(C2), row(C2), row(C2), row(C2)],
        out_shape=[jax.ShapeDtypeStruct((N, C2), F32)] * 4,
    )(accA, accB, den1, s1, Wq, bq, Wk, bk, Wv, bv, Ws, bs)


def _final_body(acc_ref, s2_ref, den_ref, out_ref):
    acc = acc_ref[0] + acc_ref[1]
    dn = jnp.sum(den_ref[...], axis=1)
    den = jnp.maximum(dn, 1e-16)[:, None]
    h = acc / den + s2_ref[...]
    avg = jnp.mean(h, axis=0, keepdims=True)
    mx = jnp.max(h, axis=0, keepdims=True)
    out_ref[...] = jnp.concatenate([avg, mx], axis=1)


def _final(acc2, s2, den2):
    return pl.pallas_call(
        _final_body,
        out_shape=jax.ShapeDtypeStruct((1, 2 * C2), F32),
    )(acc2, s2, den2)


_alpha1 = _make_alpha_kernel(C1)
_alpha2 = _make_alpha_kernel(C2)
_agg1a = _make_agg_kernel(D_IN, True, 25)
_agg1b = _make_agg_kernel(D_IN, False, 25)
_agg2 = _make_agg_kernel(C2, True, 25)


def kernel(x, edge_index, edge_weight,
           Wq1, bq1, Wk1, bk1, Wv1, bv1, Ws1, bs1,
           Wq2, bq2, Wk2, bk2, Wv2, bv2, Ws2, bs2):
    src3 = edge_index[0].reshape(NW, NCH, CH)
    dst3 = edge_index[1].reshape(NW, NCH, CH)
    r1 = lambda b: b.reshape(1, -1)

    q1, k1, va, vb, s1 = _proj1(x, Wq1, r1(bq1), Wk1, r1(bk1),
                                Wv1, r1(bv1), Ws1, r1(bs1))
    alpha1, wmax1 = _alpha1(dst3, src3, q1, k1)
    accA, den1 = _agg1a(dst3, src3, alpha1, wmax1, va)
    accB = _agg1b(dst3, src3, alpha1, wmax1, vb)
    q2, k2, v2, s2 = _proj2(accA, accB, den1.T, s1, Wq2, r1(bq2),
                            Wk2, r1(bk2), Wv2, r1(bv2), Ws2, r1(bs2))
    alpha2, wmax2 = _alpha2(dst3, src3, q2, k2)
    acc2, den2 = _agg2(dst3, src3, alpha2, wmax2, v2)
    pooled = _final(acc2, s2, den2.T)
    return pooled.reshape(2 * C2)


# speedup vs baseline: 14.0171x; 1.0773x over previous
import functools
import math

import jax
import jax.numpy as jnp
from jax import lax
from jax.experimental import pallas as pl
from jax.experimental.pallas import tpu as pltpu
from jax.experimental.pallas import tpu_sc as plsc

N = 10000
E = 320000
D_IN = 128
C1 = 256
C2 = 64

NC = 2
NS = 16
L = 16
NW = NC * NS
EPW = E // NW
CH = 80
NCH = EPW // CH

F32 = jnp.float32


def _mesh():
    return plsc.VectorSubcoreMesh(core_axis_name="c", subcore_axis_name="s")


def _make_alpha_kernel(C):
    NJ = C // L

    def body(dst3, src3, q_hbm, k_hbm, alpha_out, wmax_hbm,
             didx_all, sidx_all, qb2, kb2, pb, ab_all, mxb,
             sq0, sq1, sk0, sk1):
        cid = lax.axis_index("c")
        sid = lax.axis_index("s")
        wid = sid * NC + cid
        sq = (sq0, sq1)
        sk = (sk0, sk1)

        pltpu.sync_copy(dst3.at[wid], didx_all)
        pltpu.sync_copy(src3.at[wid], sidx_all)

        def issue(p, ch):
            pltpu.async_copy(q_hbm.at[didx_all.at[ch]], qb2.at[p], sq[p])
            pltpu.async_copy(k_hbm.at[sidx_all.at[ch]], kb2.at[p], sk[p])

        def compute(p, ch, mx):
            pltpu.make_async_copy(q_hbm.at[didx_all.at[ch]], qb2.at[p],
                                  sq[p]).wait()
            pltpu.make_async_copy(k_hbm.at[sidx_all.at[ch]], kb2.at[p],
                                  sk[p]).wait()

            def edge_body(e, carry):
                acc = qb2[p, e, pl.ds(0, L)] * kb2[p, e, pl.ds(0, L)]
                for j in range(1, NJ):
                    acc = acc + (qb2[p, e, pl.ds(j * L, L)] *
                                 kb2[p, e, pl.ds(j * L, L)])
                pb[e, pl.ds(0, L)] = acc
                return carry

            lax.fori_loop(0, CH, edge_body, 0, unroll=2)

            def group_body(g, mx_in):
                rows = g * L + lax.iota(jnp.int32, L)
                a16 = plsc.load_gather(pb, [rows, jnp.zeros((L,), jnp.int32)])
                for j in range(1, L):
                    a16 = a16 + plsc.load_gather(
                        pb, [rows, jnp.full((L,), j, jnp.int32)])
                ab_all[ch, pl.ds(g * L, L)] = a16
                return jnp.maximum(mx_in, a16)

            mx = lax.fori_loop(0, CH // L, group_body, mx)
            return mx

        mx0 = jnp.full((L,), -jnp.inf, F32)
        issue(0, 0)

        def pair(t, mx):
            issue(1, 2 * t + 1)
            mx = compute(0, 2 * t, mx)
            issue(0, 2 * t + 2)
            mx = compute(1, 2 * t + 1, mx)
            return mx

        mx = lax.fori_loop(0, NCH // 2, pair, mx0)
        mx = compute(0, NCH - 1, mx)
        pltpu.sync_copy(ab_all, alpha_out.at[wid])
        mxb[pl.ds(0, L)] = mx
        pltpu.sync_copy(mxb, wmax_hbm.at[wid])

    kern = pl.kernel(
        body,
        out_type=(jax.ShapeDtypeStruct((NW, NCH, CH), F32),
                  jax.ShapeDtypeStruct((NW, L), F32)),
        mesh=_mesh(),
        compiler_params=pltpu.CompilerParams(needs_layout_passes=False,
                                             use_tc_tiling_on_sc=False),
        scratch_types=[
            pltpu.VMEM((NCH, CH), jnp.int32),
            pltpu.VMEM((NCH, CH), jnp.int32),
            pltpu.VMEM((2, CH, C), F32),
            pltpu.VMEM((2, CH, C), F32),
            pltpu.VMEM((CH, L), F32),
            pltpu.VMEM((NCH, CH), F32),
            pltpu.VMEM((L,), F32),
            pltpu.SemaphoreType.DMA,
            pltpu.SemaphoreType.DMA,
            pltpu.SemaphoreType.DMA,
            pltpu.SemaphoreType.DMA,
        ],
    )
    return kern


def _make_agg_kernel(D, with_den, cps):
    ZR = 8
    NBLK = N // ZR
    CHA = 80
    NCHA = EPW // CHA
    NSLAB = NCHA // cps

    def body(dst3, src3, alpha3, wmax_hbm, v_hbm, *refs):
        if with_den:
            (out_hbm, den_hbm, didx_s, sidx_s, vb2, ab_s, mb, zb,
             denb, acc_sh, sv0, sv1) = refs
        else:
            (out_hbm, didx_s, sidx_s, vb2, ab_s, mb, zb,
             acc_sh, sv0, sv1) = refs
        cid = lax.axis_index("c")
        sid = lax.axis_index("s")
        wid = sid * NC + cid
        sv = (sv0, sv1)

        def zfill(r, c):
            for j in range(D // L):
                zb[r, pl.ds(j * L, L)] = jnp.zeros((L,), F32)
            return c
        lax.fori_loop(0, ZR, zfill, 0)

        def zcopy(t, c):
            blk = sid + t * NS

            @pl.when(blk < NBLK)
            def _():
                off = pl.multiple_of(blk * ZR, 8)
                pltpu.sync_copy(zb, acc_sh.at[pl.ds(off, ZR)])
            return c
        lax.fori_loop(0, (NBLK + NS - 1) // NS, zcopy, 0)

        pltpu.sync_copy(wmax_hbm, mb)

        def mred(i, m):
            return jnp.maximum(m, mb[i, pl.ds(0, L)])
        m16 = lax.fori_loop(0, NW, mred, jnp.full((L,), -jnp.inf, F32))
        gmax = jnp.max(m16)

        if with_den:
            def dzero(i, c):
                denb[pl.ds(i * L, L)] = jnp.zeros((L,), F32)
                return c
            lax.fori_loop(0, N // L, dzero, 0)

        plsc.subcore_barrier()

        def issue(p, j):
            pltpu.async_copy(v_hbm.at[sidx_s.at[j]], vb2.at[p], sv[p])

        def compute(p, j):
            pltpu.make_async_copy(v_hbm.at[sidx_s.at[j]], vb2.at[p],
                                  sv[p]).wait()

            def group_body(g, c):
                a16 = ab_s[j, pl.ds(g * L, L)]
                e16 = jnp.exp(a16 - gmax)
                if with_den:
                    d16 = didx_s[j, pl.ds(g * L, L)]
                    plsc.addupdate_scatter(denb, [d16], e16)
                for l in range(L):
                    s = e16[l]
                    e = g * L + l
                    for jj in range(D // L):
                        vb2[p, e, pl.ds(jj * L, L)] = (
                            vb2[p, e, pl.ds(jj * L, L)] * s)
                return c
            lax.fori_loop(0, CHA // L, group_body, 0)

            pltpu.sync_copy(vb2.at[p], acc_sh.at[didx_s.at[j]], add=True)

        def slab(si, c):
            pltpu.sync_copy(dst3.at[wid, pl.ds(si * cps, cps)], didx_s)
            pltpu.sync_copy(src3.at[wid, pl.ds(si * cps, cps)], sidx_s)
            pltpu.sync_copy(alpha3.at[wid, pl.ds(si * cps, cps)], ab_s)
            issue(0, 0)

            def pair(t, cc):
                issue(1, 2 * t + 1)
                compute(0, 2 * t)
                issue(0, 2 * t + 2)
                compute(1, 2 * t + 1)
                return cc
            lax.fori_loop(0, cps // 2, pair, 0)
            compute(0, cps - 1)
            return c

        lax.fori_loop(0, NSLAB, slab, 0)

        plsc.subcore_barrier()

        @pl.when(sid == 0)
        def _():
            pltpu.sync_copy(acc_sh, out_hbm.at[cid])
        if with_den:
            pltpu.sync_copy(denb, den_hbm.at[wid])

    out_type = [jax.ShapeDtypeStruct((NC, N, D), F32)]
    if with_den:
        out_type.append(jax.ShapeDtypeStruct((NW, N), F32))
    scratch = [
        pltpu.VMEM((cps, CHA), jnp.int32),
        pltpu.VMEM((cps, CHA), jnp.int32),
        pltpu.VMEM((2, CHA, D), F32),
        pltpu.VMEM((cps, CHA), F32),
        pltpu.VMEM((NW, L), F32),
        pltpu.VMEM((ZR, D), F32),
    ]
    if with_den:
        scratch.append(pltpu.VMEM((N,), F32))
    scratch += [
        pltpu.VMEM_SHARED((N, D), F32),
        pltpu.SemaphoreType.DMA,
        pltpu.SemaphoreType.DMA,
    ]
    kern = pl.kernel(
        body,
        out_type=tuple(out_type) if with_den else out_type[0],
        mesh=_mesh(),
        compiler_params=pltpu.CompilerParams(needs_layout_passes=False,
                                             use_tc_tiling_on_sc=False),
        scratch_types=scratch,
    )
    return kern


_RB = 1000


def _proj1_body(x_ref, wq, bq, wk, bk, wv, bv, ws, bs,
                q_ref, k_ref, va_ref, vb_ref, s_ref):
    xb = jnp.nan_to_num(x_ref[...], nan=0.0)
    scale = 1.0 / math.sqrt(C1)
    q = (jnp.dot(xb, wq[...], preferred_element_type=F32) + bq[...]) * scale
    k = jnp.dot(xb, wk[...], preferred_element_type=F32) + bk[...]
    v = jnp.dot(xb, wv[...], preferred_element_type=F32) + bv[...]
    s = jnp.dot(xb, ws[...], preferred_element_type=F32) + bs[...]
    q_ref[...] = q
    k_ref[...] = k
    va_ref[...] = v[:, :D_IN]
    vb_ref[...] = v[:, D_IN:]
    s_ref[...] = s


def _proj1(x, Wq, bq, Wk, bk, Wv, bv, Ws, bs):
    grid = N // _RB
    full = lambda r, c: pl.BlockSpec((r, c), lambda i: (0, 0))
    row = lambda c: pl.BlockSpec((_RB, c), lambda i: (i, 0))
    return pl.pallas_call(
        _proj1_body,
        grid=(grid,),
        in_specs=[row(D_IN),
                  full(D_IN, C1), full(1, C1), full(D_IN, C1), full(1, C1),
                  full(D_IN, C1), full(1, C1), full(D_IN, C1), full(1, C1)],
        out_specs=[row(C1), row(C1), row(D_IN), row(D_IN), row(C1)],
        out_shape=[jax.ShapeDtypeStruct((N, C1), F32),
                   jax.ShapeDtypeStruct((N, C1), F32),
                   jax.ShapeDtypeStruct((N, D_IN), F32),
                   jax.ShapeDtypeStruct((N, D_IN), F32),
                   jax.ShapeDtypeStruct((N, C1), F32)],
    )(x, Wq, bq, Wk, bk, Wv, bv, Ws, bs)


def _proj2_body(aa_ref, ab_ref, den_ref, s1_ref, wq, bq, wk, bk, wv, bv,
                ws, bs, q_ref, k_ref, v_ref, s_ref):
    accA = aa_ref[0] + aa_ref[1]
    accB = ab_ref[0] + ab_ref[1]
    dn = jnp.sum(den_ref[...], axis=1)
    den = jnp.maximum(dn, 1e-16)[:, None]
    h = jnp.concatenate([accA, accB], axis=1) / den + s1_ref[...]
    scale = 1.0 / math.sqrt(C2)
    q = (jnp.dot(h, wq[...], preferred_element_type=F32) + bq[...]) * scale
    k = jnp.dot(h, wk[...], preferred_element_type=F32) + bk[...]
    v = jnp.dot(h, wv[...], preferred_element_type=F32) + bv[...]
    s = jnp.dot(h, ws[...], preferred_element_type=F32) + bs[...]
    q_ref[...] = q
    k_ref[...] = k
    v_ref[...] = v
    s_ref[...] = s


def _proj2(accA, accB, den1, s1, Wq, bq, Wk, bk, Wv, bv, Ws, bs):
    grid = N // _RB
    full = lambda r, c: pl.BlockSpec((r, c), lambda i: (0, 0))
    row = lambda c: pl.BlockSpec((_RB, c), lambda i: (i, 0))
    acc = lambda c: pl.BlockSpec((NC, _RB, c), lambda i: (0, i, 0))
    dspec = pl.BlockSpec((_RB, NW), lambda i: (i, 0))
    return pl.pallas_call(
        _proj2_body,
        grid=(grid,),
        in_specs=[acc(D_IN), acc(D_IN), dspec, row(C1),
                  full(C1, C2), full(1, C2), full(C1, C2), full(1, C2),
                  full(C1, D_IN), full(1, D_IN), full(C1, C2), full(1, C2)],
        out_specs=[row(C2), row(C2), row(D_IN), row(C2)],
        out_shape=[jax.ShapeDtypeStruct((N, C2), F32),
                   jax.ShapeDtypeStruct((N, C2), F32),
                   jax.ShapeDtypeStruct((N, D_IN), F32),
                   jax.ShapeDtypeStruct((N, C2), F32)],
    )(accA, accB, den1, s1, Wq, bq, Wk, bk, Wv, bv, Ws, bs)


def _final_body(acc_ref, s2_ref, den_ref, out_ref):
    acc = acc_ref[0] + acc_ref[1]
    dn = jnp.sum(den_ref[...], axis=1)
    den = jnp.maximum(dn, 1e-16)[:, None]
    h = acc[:, :C2] / den + s2_ref[...]
    avg = jnp.mean(h, axis=0, keepdims=True)
    mx = jnp.max(h, axis=0, keepdims=True)
    out_ref[...] = jnp.concatenate([avg, mx], axis=1)


def _final(acc2, s2, den2):
    return pl.pallas_call(
        _final_body,
        out_shape=jax.ShapeDtypeStruct((1, 2 * C2), F32),
    )(acc2, s2, den2)


_alpha1 = _make_alpha_kernel(C1)
_alpha2 = _make_alpha_kernel(C2)
_agg1a = _make_agg_kernel(D_IN, True, 25)
_agg1b = _make_agg_kernel(D_IN, False, 25)
_agg2 = _make_agg_kernel(D_IN, True, 25)


def kernel(x, edge_index, edge_weight,
           Wq1, bq1, Wk1, bk1, Wv1, bv1, Ws1, bs1,
           Wq2, bq2, Wk2, bk2, Wv2, bv2, Ws2, bs2):
    src3 = edge_index[0].reshape(NW, NCH, CH)
    dst3 = edge_index[1].reshape(NW, NCH, CH)
    r1 = lambda b: b.reshape(1, -1)

    q1, k1, va, vb, s1 = _proj1(x, Wq1, r1(bq1), Wk1, r1(bk1),
                                Wv1, r1(bv1), Ws1, r1(bs1))
    alpha1, wmax1 = _alpha1(dst3, src3, q1, k1)
    accA, den1 = _agg1a(dst3, src3, alpha1, wmax1, va)
    accB = _agg1b(dst3, src3, alpha1, wmax1, vb)
    Wv2p = jnp.pad(Wv2, ((0, 0), (0, D_IN - C2)))
    bv2p = jnp.pad(bv2, (0, D_IN - C2))
    q2, k2, v2, s2 = _proj2(accA, accB, den1.T, s1, Wq2, r1(bq2),
                            Wk2, r1(bk2), Wv2p, r1(bv2p), Ws2, r1(bs2))
    alpha2, wmax2 = _alpha2(dst3, src3, q2, k2)
    acc2, den2 = _agg2(dst3, src3, alpha2, wmax2, v2)
    pooled = _final(acc2, s2, den2.T)
    return pooled.reshape(2 * C2)


# speedup vs baseline: 14.0234x; 1.0005x over previous
import math

import jax
import jax.numpy as jnp
from jax import lax
from jax.experimental import pallas as pl
from jax.experimental.pallas import tpu as pltpu
from jax.experimental.pallas import tpu_sc as plsc

N = 10000
E = 320000
D_IN = 128
C1 = 256
C2 = 64

NC = 2
NS = 16
L = 16
NW = NC * NS
EPW = E // NW
CH = 80
NCH = EPW // CH

F32 = jnp.float32


def _mesh():
    return plsc.VectorSubcoreMesh(core_axis_name="c", subcore_axis_name="s")


def _make_alpha_kernel(C):
    NJ = C // L

    def body(dst3, src3, q_hbm, k_hbm, alpha_out, wmax_hbm,
             didx_all, sidx_all, qb2, kb2, pb, ab_all, mxb,
             sq0, sq1, sk0, sk1):
        cid = lax.axis_index("c")
        sid = lax.axis_index("s")
        wid = sid * NC + cid
        sq = (sq0, sq1)
        sk = (sk0, sk1)

        pltpu.sync_copy(dst3.at[wid], didx_all)
        pltpu.sync_copy(src3.at[wid], sidx_all)

        def issue(p, ch):
            pltpu.async_copy(q_hbm.at[didx_all.at[ch]], qb2.at[p], sq[p])
            pltpu.async_copy(k_hbm.at[sidx_all.at[ch]], kb2.at[p], sk[p])

        def compute(p, ch, mx):
            pltpu.make_async_copy(q_hbm.at[didx_all.at[ch]], qb2.at[p],
                                  sq[p]).wait()
            pltpu.make_async_copy(k_hbm.at[sidx_all.at[ch]], kb2.at[p],
                                  sk[p]).wait()

            def edge_body(e, carry):
                acc = qb2[p, e, pl.ds(0, L)] * kb2[p, e, pl.ds(0, L)]
                for j in range(1, NJ):
                    acc = acc + (qb2[p, e, pl.ds(j * L, L)] *
                                 kb2[p, e, pl.ds(j * L, L)])
                pb[e, pl.ds(0, L)] = acc
                return carry

            lax.fori_loop(0, CH, edge_body, 0, unroll=2)

            def group_body(g, mx_in):
                rows = g * L + lax.iota(jnp.int32, L)
                a16 = plsc.load_gather(pb, [rows, jnp.zeros((L,), jnp.int32)])
                for j in range(1, L):
                    a16 = a16 + plsc.load_gather(
                        pb, [rows, jnp.full((L,), j, jnp.int32)])
                ab_all[ch, pl.ds(g * L, L)] = a16
                return jnp.maximum(mx_in, a16)

            mx = lax.fori_loop(0, CH // L, group_body, mx)
            return mx

        mx0 = jnp.full((L,), -jnp.inf, F32)
        issue(0, 0)

        def pair(t, mx):
            issue(1, 2 * t + 1)
            mx = compute(0, 2 * t, mx)
            issue(0, 2 * t + 2)
            mx = compute(1, 2 * t + 1, mx)
            return mx

        mx = lax.fori_loop(0, NCH // 2, pair, mx0)
        mx = compute(0, NCH - 1, mx)
        pltpu.sync_copy(ab_all, alpha_out.at[wid])
        mxb[pl.ds(0, L)] = mx
        pltpu.sync_copy(mxb, wmax_hbm.at[wid])

    kern = pl.kernel(
        body,
        out_type=(jax.ShapeDtypeStruct((NW, NCH, CH), F32),
                  jax.ShapeDtypeStruct((NW, L), F32)),
        mesh=_mesh(),
        compiler_params=pltpu.CompilerParams(needs_layout_passes=False,
                                             use_tc_tiling_on_sc=False),
        scratch_types=[
            pltpu.VMEM((NCH, CH), jnp.int32),
            pltpu.VMEM((NCH, CH), jnp.int32),
            pltpu.VMEM((2, CH, C), F32),
            pltpu.VMEM((2, CH, C), F32),
            pltpu.VMEM((CH, L), F32),
            pltpu.VMEM((NCH, CH), F32),
            pltpu.VMEM((L,), F32),
            pltpu.SemaphoreType.DMA,
            pltpu.SemaphoreType.DMA,
            pltpu.SemaphoreType.DMA,
            pltpu.SemaphoreType.DMA,
        ],
    )
    return kern


def _make_agg_kernel(D, with_den, cps):
    ZR = 8
    NBLK = N // ZR
    CHA = 80
    NCHA = EPW // CHA
    NSLAB = NCHA // cps

    def body(dst3, src3, alpha3, wmax_hbm, v_hbm, *refs):
        if with_den:
            (out_hbm, den_hbm, didx_s, sidx_s, vb2, ab_s, mb, zb,
             denb, acc_sh, sv0, sv1) = refs
        else:
            (out_hbm, didx_s, sidx_s, vb2, ab_s, mb, zb,
             acc_sh, sv0, sv1) = refs
        cid = lax.axis_index("c")
        sid = lax.axis_index("s")
        wid = sid * NC + cid
        sv = (sv0, sv1)

        def zfill(r, c):
            for j in range(D // L):
                zb[r, pl.ds(j * L, L)] = jnp.zeros((L,), F32)
            return c
        lax.fori_loop(0, ZR, zfill, 0)

        def zcopy(t, c):
            blk = sid + t * NS

            @pl.when(blk < NBLK)
            def _():
                off = pl.multiple_of(blk * ZR, 8)
                pltpu.sync_copy(zb, acc_sh.at[pl.ds(off, ZR)])
            return c
        lax.fori_loop(0, (NBLK + NS - 1) // NS, zcopy, 0)

        pltpu.sync_copy(wmax_hbm, mb)

        def mred(i, m):
            return jnp.maximum(m, mb[i, pl.ds(0, L)])
        m16 = lax.fori_loop(0, NW, mred, jnp.full((L,), -jnp.inf, F32))
        gmax = jnp.max(m16)

        if with_den:
            def dzero(i, c):
                denb[pl.ds(i * L, L)] = jnp.zeros((L,), F32)
                return c
            lax.fori_loop(0, N // L, dzero, 0)

        plsc.subcore_barrier()

        def issue(p, j):
            pltpu.async_copy(v_hbm.at[sidx_s.at[j]], vb2.at[p], sv[p])

        def compute(p, j):
            pltpu.make_async_copy(v_hbm.at[sidx_s.at[j]], vb2.at[p],
                                  sv[p]).wait()

            def group_body(g, c):
                a16 = ab_s[j, pl.ds(g * L, L)]
                e16 = jnp.exp(a16 - gmax)
                if with_den:
                    d16 = didx_s[j, pl.ds(g * L, L)]
                    plsc.addupdate_scatter(denb, [d16], e16)
                for l in range(L):
                    s = e16[l]
                    e = g * L + l
                    for jj in range(D // L):
                        vb2[p, e, pl.ds(jj * L, L)] = (
                            vb2[p, e, pl.ds(jj * L, L)] * s)
                return c
            lax.fori_loop(0, CHA // L, group_body, 0)

            pltpu.sync_copy(vb2.at[p], acc_sh.at[didx_s.at[j]], add=True)

        def slab(si, c):
            pltpu.sync_copy(dst3.at[wid, pl.ds(si * cps, cps)], didx_s)
            pltpu.sync_copy(src3.at[wid, pl.ds(si * cps, cps)], sidx_s)
            pltpu.sync_copy(alpha3.at[wid, pl.ds(si * cps, cps)], ab_s)
            issue(0, 0)

            def pair(t, cc):
                issue(1, 2 * t + 1)
                compute(0, 2 * t)
                issue(0, 2 * t + 2)
                compute(1, 2 * t + 1)
                return cc
            lax.fori_loop(0, cps // 2, pair, 0)
            compute(0, cps - 1)
            return c

        lax.fori_loop(0, NSLAB, slab, 0)

        plsc.subcore_barrier()

        @pl.when(sid == 0)
        def _():
            pltpu.sync_copy(acc_sh, out_hbm.at[cid])
        if with_den:
            pltpu.sync_copy(denb, den_hbm.at[wid])

    out_type = [jax.ShapeDtypeStruct((NC, N, D), F32)]
    if with_den:
        out_type.append(jax.ShapeDtypeStruct((NW, N), F32))
    scratch = [
        pltpu.VMEM((cps, CHA), jnp.int32),
        pltpu.VMEM((cps, CHA), jnp.int32),
        pltpu.VMEM((2, CHA, D), F32),
        pltpu.VMEM((cps, CHA), F32),
        pltpu.VMEM((NW, L), F32),
        pltpu.VMEM((ZR, D), F32),
    ]
    if with_den:
        scratch.append(pltpu.VMEM((N,), F32))
    scratch += [
        pltpu.VMEM_SHARED((N, D), F32),
        pltpu.SemaphoreType.DMA,
        pltpu.SemaphoreType.DMA,
    ]
    kern = pl.kernel(
        body,
        out_type=tuple(out_type) if with_den else out_type[0],
        mesh=_mesh(),
        compiler_params=pltpu.CompilerParams(needs_layout_passes=False,
                                             use_tc_tiling_on_sc=False),
        scratch_types=scratch,
    )
    return kern


_RB = 1000


def _proj1_body(x_ref, wq, bq, wk, bk, wv, bv, ws, bs,
                q_ref, k_ref, va_ref, vb_ref, s_ref):
    xb = jnp.nan_to_num(x_ref[...], nan=0.0)
    scale = 1.0 / math.sqrt(C1)
    q = (jnp.dot(xb, wq[...], preferred_element_type=F32) + bq[...]) * scale
    k = jnp.dot(xb, wk[...], preferred_element_type=F32) + bk[...]
    v = jnp.dot(xb, wv[...], preferred_element_type=F32) + bv[...]
    s = jnp.dot(xb, ws[...], preferred_element_type=F32) + bs[...]
    q_ref[...] = q
    k_ref[...] = k
    va_ref[...] = v[:, :D_IN]
    vb_ref[...] = v[:, D_IN:]
    s_ref[...] = s


def _proj1(x, Wq, bq, Wk, bk, Wv, bv, Ws, bs):
    grid = N // _RB
    full = lambda r, c: pl.BlockSpec((r, c), lambda i: (0, 0))
    row = lambda c: pl.BlockSpec((_RB, c), lambda i: (i, 0))
    return pl.pallas_call(
        _proj1_body,
        grid=(grid,),
        in_specs=[row(D_IN),
                  full(D_IN, C1), full(1, C1), full(D_IN, C1), full(1, C1),
                  full(D_IN, C1), full(1, C1), full(D_IN, C1), full(1, C1)],
        out_specs=[row(C1), row(C1), row(D_IN), row(D_IN), row(C1)],
        out_shape=[jax.ShapeDtypeStruct((N, C1), F32),
                   jax.ShapeDtypeStruct((N, C1), F32),
                   jax.ShapeDtypeStruct((N, D_IN), F32),
                   jax.ShapeDtypeStruct((N, D_IN), F32),
                   jax.ShapeDtypeStruct((N, C1), F32)],
    )(x, Wq, bq, Wk, bk, Wv, bv, Ws, bs)


def _proj2_body(aa_ref, ab_ref, den_ref, s1_ref, wq, bq, wk, bk, wv, bv,
                ws, bs, q_ref, k_ref, v_ref, s_ref):
    accA = aa_ref[0] + aa_ref[1]
    accB = ab_ref[0] + ab_ref[1]
    dn = jnp.sum(den_ref[...], axis=1)
    den = jnp.maximum(dn, 1e-16)[:, None]
    h = jnp.concatenate([accA, accB], axis=1) / den + s1_ref[...]
    scale = 1.0 / math.sqrt(C2)
    q = (jnp.dot(h, wq[...], preferred_element_type=F32) + bq[...]) * scale
    k = jnp.dot(h, wk[...], preferred_element_type=F32) + bk[...]
    v = jnp.dot(h, wv[...], preferred_element_type=F32) + bv[...]
    s = jnp.dot(h, ws[...], preferred_element_type=F32) + bs[...]
    q_ref[...] = q
    k_ref[...] = k
    v_ref[...] = v
    s_ref[...] = s


def _proj2(accA, accB, den1, s1, Wq, bq, Wk, bk, Wv, bv, Ws, bs):
    grid = N // _RB
    full = lambda r, c: pl.BlockSpec((r, c), lambda i: (0, 0))
    row = lambda c: pl.BlockSpec((_RB, c), lambda i: (i, 0))
    acc = lambda c: pl.BlockSpec((NC, _RB, c), lambda i: (0, i, 0))
    dspec = pl.BlockSpec((_RB, NW), lambda i: (i, 0))
    return pl.pallas_call(
        _proj2_body,
        grid=(grid,),
        in_specs=[acc(D_IN), acc(D_IN), dspec, row(C1),
                  full(C1, C2), full(1, C2), full(C1, C2), full(1, C2),
                  full(C1, D_IN), full(1, D_IN), full(C1, C2), full(1, C2)],
        out_specs=[row(C2), row(C2), row(D_IN), row(C2)],
        out_shape=[jax.ShapeDtypeStruct((N, C2), F32),
                   jax.ShapeDtypeStruct((N, C2), F32),
                   jax.ShapeDtypeStruct((N, D_IN), F32),
                   jax.ShapeDtypeStruct((N, C2), F32)],
    )(accA, accB, den1, s1, Wq, bq, Wk, bk, Wv, bv, Ws, bs)


def _final_body(acc_ref, s2_ref, den_ref, out_ref):
    acc = acc_ref[0] + acc_ref[1]
    dn = jnp.sum(den_ref[...], axis=1)
    den = jnp.maximum(dn, 1e-16)[:, None]
    h = acc[:, :C2] / den + s2_ref[...]
    avg = jnp.mean(h, axis=0, keepdims=True)
    mx = jnp.max(h, axis=0, keepdims=True)
    out_ref[...] = jnp.concatenate([avg, mx], axis=1)


def _final(acc2, s2, den2):
    return pl.pallas_call(
        _final_body,
        out_shape=jax.ShapeDtypeStruct((1, 2 * C2), F32),
    )(acc2, s2, den2)


_alpha1 = _make_alpha_kernel(C1)
_alpha2 = _make_alpha_kernel(C2)
_agg1a = _make_agg_kernel(D_IN, True, 25)
_agg1b = _make_agg_kernel(D_IN, False, 25)
_agg2 = _make_agg_kernel(D_IN, True, 25)


def kernel(x, edge_index, edge_weight,
           Wq1, bq1, Wk1, bk1, Wv1, bv1, Ws1, bs1,
           Wq2, bq2, Wk2, bk2, Wv2, bv2, Ws2, bs2):
    src3 = edge_index[0].reshape(NW, NCH, CH)
    dst3 = edge_index[1].reshape(NW, NCH, CH)
    r1 = lambda b: b.reshape(1, -1)

    q1, k1, va, vb, s1 = _proj1(x, Wq1, r1(bq1), Wk1, r1(bk1),
                                Wv1, r1(bv1), Ws1, r1(bs1))
    alpha1, wmax1 = _alpha1(dst3, src3, q1, k1)
    accA, den1 = _agg1a(dst3, src3, alpha1, wmax1, va)
    accB = _agg1b(dst3, src3, alpha1, wmax1, vb)
    Wv2p = jnp.pad(Wv2, ((0, 0), (0, D_IN - C2)))
    bv2p = jnp.pad(bv2, (0, D_IN - C2))
    q2, k2, v2, s2 = _proj2(accA, accB, den1.T, s1, Wq2, r1(bq2),
                            Wk2, r1(bk2), Wv2p, r1(bv2p), Ws2, r1(bs2))
    alpha2, wmax2 = _alpha2(dst3, src3, q2, k2)
    acc2, den2 = _agg2(dst3, src3, alpha2, wmax2, v2)
    pooled = _final(acc2, s2, den2.T)
    return pooled.reshape(2 * C2)
